# Initial kernel scaffold; baseline (speedup 1.0000x reference)
import jax
import jax.numpy as jnp
from jax.experimental import pallas as pl


def kernel(obs_batch, true_lens, log_T, log_pi):
    raise NotImplementedError("write your pallas kernel here")


# speedup vs baseline: 49.8302x; 49.8302x over previous
import jax
import jax.numpy as jnp
from jax.experimental import pallas as pl
from jax.experimental.pallas import tpu as pltpu

N_OBS = 100
C = 32
N_STATES = N_OBS * C
B = 16
T_LEN = 512
NEG_INF = float("-inf")


def _logaddexp(a, b):
    mx = jnp.maximum(a, b)
    mn = jnp.minimum(a, b)
    r = mx + jnp.log1p(jnp.exp(mn - mx))
    return jnp.where(jnp.isfinite(mx), r, mx)


def _lse_rows(x):
    m = jnp.max(x, axis=1, keepdims=True)
    return m + jnp.log(jnp.sum(jnp.exp(x - m), axis=1, keepdims=True))


def _main_body(obs_ref, c4_ref, tl_ref, gmod_ref, log_T_ref, log_pi_ref,
               z_ref, acc_ref, g_ref,
               las_ref, buf_ref, sem_ref):
    def init_body(i, _):
        acc_ref[pl.ds(i * 100, 100)] = jnp.full((100, C, 128), NEG_INF,
                                                jnp.float32)
        return 0
    jax.lax.fori_loop(0, 25, init_body, 0)
    g_ref[...] = jnp.full((N_OBS, C), NEG_INF, jnp.float32)

    gid = jax.lax.broadcasted_iota(jnp.int32, (B, 128), 1) // C

    def fetch(slot, t0, t1):
        for b in range(B):
            r = obs_ref[b, t0]
            c4 = c4_ref[b, t1]
            pltpu.make_async_copy(
                log_T_ref.at[pl.ds(r * C, C), pl.ds(c4 * 128, 128)],
                buf_ref.at[slot, b],
                sem_ref.at[slot, b],
            ).start()

    def wait(slot):
        for b in range(B):
            pltpu.make_async_copy(
                log_T_ref.at[pl.ds(0, C), pl.ds(0, 128)],
                buf_ref.at[slot, b],
                sem_ref.at[slot, b],
            ).wait()

    la0 = jnp.stack([log_pi_ref[obs_ref[b, 0]] for b in range(B)])
    las_ref[0] = la0

    fetch(1, 0, 1)

    def fwd_body(t, la):
        s = jax.lax.rem(t, 2)

        @pl.when(t + 1 < T_LEN)
        def _():
            fetch(1 - s, t, t + 1)

        wait(s)
        mask = gid == gmod_ref[t][:, None]
        Tb = buf_ref[s]
        Em = jnp.where(mask[:, None, :], jnp.exp(Tb), 0.0)
        m = jnp.max(la, axis=1, keepdims=True)
        v = jnp.exp(la - m)
        ssum = jnp.sum(v[:, :, None] * Em, axis=1)
        s4 = (ssum[:, 0:32] + ssum[:, 32:64]
              + ssum[:, 64:96] + ssum[:, 96:128])
        lan = m + jnp.log(s4)
        las_ref[t] = lan
        return lan

    jax.lax.fori_loop(1, T_LEN, fwd_body, la0, unroll=False)

    las_last = jnp.stack([las_ref[tl_ref[b] - 1, b] for b in range(B)])
    log_Z = _lse_rows(las_last)
    z_ref[0, 0] = jnp.sum(log_Z)

    fetch(0, T_LEN - 2, T_LEN - 1)

    def bwd_body(k, lb):
        t = T_LEN - 2 - k
        s = jax.lax.rem(t, 2)

        @pl.when(t > 0)
        def _():
            fetch(1 - s, t - 1, t)

        wait(s)
        mask = gid == gmod_ref[t + 1][:, None]
        Tb = buf_ref[s]
        las_t = las_ref[t]
        lbt = jnp.concatenate([lb, lb, lb, lb], axis=1)
        lxs = (las_t[:, :, None] + Tb + lbt[:, None, :]
               - log_Z[:, :, None])

        def rmw(b):
            @pl.when(t < tl_ref[b] - 1)
            def _():
                q = obs_ref[b, t] * 25 + c4_ref[b, t + 1]
                cur = acc_ref[q]
                acc_ref[q] = jnp.where(mask[b][None, :],
                                       _logaddexp(cur, lxs[b]), cur)
        for b in range(B):
            rmw(b)

        Em = jnp.where(mask[:, None, :], jnp.exp(Tb), 0.0)
        m = jnp.max(lb, axis=1, keepdims=True)
        w = jnp.exp(lb - m)
        wt = jnp.concatenate([w, w, w, w], axis=1)
        lbc = m + jnp.log(jnp.sum(Em * wt[:, None, :], axis=2))
        return lbc

    lb0 = jax.lax.fori_loop(0, T_LEN - 1, bwd_body,
                            jnp.zeros((B, C), jnp.float32), unroll=False)

    lg0 = las_ref[0] + lb0 - log_Z

    def gacc(b):
        o0 = obs_ref[b, 0]
        g_ref[o0] = _logaddexp(g_ref[o0], lg0[b])
    for b in range(B):
        gacc(b)


def _assemble_body(acc_ref, out_ref):
    out_ref[...] = acc_ref[0]


@jax.jit
def kernel(obs_batch, true_lens, log_T, log_pi):
    log_pi_2d = log_pi.reshape(N_OBS, C)
    c4 = obs_batch // 4
    gmod = (obs_batch % 4).T

    z, acc, g = pl.pallas_call(
        _main_body,
        grid=(),
        in_specs=[
            pl.BlockSpec(memory_space=pltpu.MemorySpace.SMEM),
            pl.BlockSpec(memory_space=pltpu.MemorySpace.SMEM),
            pl.BlockSpec(memory_space=pltpu.MemorySpace.SMEM),
            pl.BlockSpec(memory_space=pltpu.MemorySpace.VMEM),
            pl.BlockSpec(memory_space=pltpu.MemorySpace.HBM),
            pl.BlockSpec(memory_space=pltpu.MemorySpace.VMEM),
        ],
        out_specs=[
            pl.BlockSpec(memory_space=pltpu.MemorySpace.SMEM),
            pl.BlockSpec(memory_space=pltpu.MemorySpace.VMEM),
            pl.BlockSpec(memory_space=pltpu.MemorySpace.VMEM),
        ],
        out_shape=[
            jax.ShapeDtypeStruct((1, 1), jnp.float32),
            jax.ShapeDtypeStruct((2500, C, 128), jnp.float32),
            jax.ShapeDtypeStruct((N_OBS, C), jnp.float32),
        ],
        scratch_shapes=[
            pltpu.VMEM((T_LEN, B, C), jnp.float32),
            pltpu.VMEM((2, B, C, 128), jnp.float32),
            pltpu.SemaphoreType.DMA((2, B)),
        ],
        compiler_params=pltpu.CompilerParams(
            vmem_limit_bytes=100 * 1024 * 1024,
        ),
    )(obs_batch, c4, true_lens, gmod, log_T, log_pi_2d)

    log_xi_sum = pl.pallas_call(
        _assemble_body,
        grid=(2500,),
        in_specs=[pl.BlockSpec((1, C, 128), lambda p: (p, 0, 0))],
        out_specs=pl.BlockSpec((C, 128), lambda p: (p // 25, p % 25)),
        out_shape=jax.ShapeDtypeStruct((N_STATES, N_STATES), jnp.float32),
    )(acc)

    return (z[0, 0], log_xi_sum, g.reshape(N_STATES))
